# baseline (speedup 1.0000x reference)
import functools

import jax
import jax.numpy as jnp
from jax import lax
from jax.experimental import pallas as pl
from jax.experimental.pallas import tpu as pltpu
from jax.experimental.pallas import tpu_sc as plsc

_N = 256
_DT = 0.05
_LANES = 16


def _make_kernel(num_rows: int, num_workers: int):
    rows = num_rows // num_workers
    assert rows % _LANES == 0 and rows * num_workers == num_rows
    steps = rows // _LANES
    mesh = plsc.VectorSubcoreMesh(core_axis_name="c", subcore_axis_name="s")
    nc = mesh.num_cores

    @functools.partial(
        pl.kernel,
        out_type=jax.ShapeDtypeStruct((num_rows, 2), jnp.float32),
        mesh=mesh,
        scratch_types=[
            pltpu.VMEM((rows, 2), jnp.float32),
            pltpu.VMEM((_N,), jnp.float32),
        ],
        compiler_params=pltpu.CompilerParams(
            needs_layout_passes=False, use_tc_tiling_on_sc=False
        ),
    )
    def run(x_hbm, f_hbm, out_hbm, buf, ftab):
        wid = lax.axis_index("s") * nc + lax.axis_index("c")
        r0 = wid * rows
        pltpu.sync_copy(f_hbm, ftab)
        pltpu.sync_copy(x_hbm.at[pl.ds(r0, rows), :], buf)

        row0 = lax.iota(jnp.int32, _LANES)
        zeros = row0 * 0
        ones = zeros + 1

        def step(i, carry):
            pr = row0 + i * _LANES
            xg = plsc.load_gather(buf, [pr, zeros])
            t = xg * 128.0 + 128.5
            t = jnp.minimum(jnp.maximum(t, 0.0), 255.0)
            idx = t.astype(jnp.int32)
            fv = plsc.load_gather(ftab, [idx])
            plsc.addupdate_scatter(buf, [pr, ones], fv * _DT)
            return carry

        lax.fori_loop(0, steps, step, 0)
        pltpu.sync_copy(buf, out_hbm.at[pl.ds(r0, rows), :])

    return run


def kernel(X, force):
    b = X.shape[0]
    return _make_kernel(b, 32)(X.astype(jnp.float32), force.astype(jnp.float32))


# speedup vs baseline: 16.1400x; 16.1400x over previous
import functools

import jax
import jax.numpy as jnp
from jax import lax
from jax.experimental import pallas as pl
from jax.experimental.pallas import tpu as pltpu
from jax.experimental.pallas import tpu_sc as plsc

_N = 256
_DT = 0.05
_LANES = 16
_BLK = 128


def _make_kernel(num_blocks: int, num_workers: int):
    wblocks = num_blocks // num_workers
    assert wblocks * num_workers == num_blocks
    mesh = plsc.VectorSubcoreMesh(core_axis_name="c", subcore_axis_name="s")
    nc = mesh.num_cores
    groups = _BLK // _LANES

    @functools.partial(
        pl.kernel,
        out_type=jax.ShapeDtypeStruct((num_blocks, 2, _BLK), jnp.float32),
        mesh=mesh,
        scratch_types=[
            pltpu.VMEM((wblocks, 2, _BLK), jnp.float32),
            pltpu.VMEM((_N,), jnp.float32),
        ],
        compiler_params=pltpu.CompilerParams(
            needs_layout_passes=False, use_tc_tiling_on_sc=False
        ),
    )
    def run(x_hbm, f_hbm, out_hbm, buf, ftab):
        wid = lax.axis_index("s") * nc + lax.axis_index("c")
        b0 = wid * wblocks
        pltpu.sync_copy(f_hbm, ftab)
        pltpu.sync_copy(x_hbm.at[pl.ds(b0, wblocks)], buf)

        def step(t, carry):
            for g in range(groups):
                xg = buf[t, 0, pl.ds(g * _LANES, _LANES)]
                s = xg * 128.0 + 128.5
                s = jnp.minimum(jnp.maximum(s, 0.0), 255.0)
                idx = s.astype(jnp.int32)
                fv = plsc.load_gather(ftab, [idx])
                vs = buf.at[t, 1, pl.ds(g * _LANES, _LANES)]
                plsc.addupdate(vs, fv * _DT)
            return carry

        lax.fori_loop(0, wblocks, step, 0)
        pltpu.sync_copy(buf, out_hbm.at[pl.ds(b0, wblocks)])

    return run


def kernel(X, force):
    b = X.shape[0]
    xt = jnp.transpose(jnp.reshape(X, (b // _BLK, _BLK, 2)), (0, 2, 1))
    yt = _make_kernel(b // _BLK, 32)(xt, force.astype(jnp.float32))
    return jnp.reshape(jnp.transpose(yt, (0, 2, 1)), (b, 2))


# speedup vs baseline: 17.3950x; 1.0778x over previous
import functools

import jax
import jax.numpy as jnp
from jax import lax
from jax.experimental import pallas as pl
from jax.experimental.pallas import tpu as pltpu
from jax.experimental.pallas import tpu_sc as plsc

_N = 256
_DT = 0.05
_LANES = 16
_BLK = 128


def _make_kernel(num_blocks: int, num_workers: int):
    wblocks = num_blocks // num_workers
    assert wblocks * num_workers == num_blocks
    mesh = plsc.VectorSubcoreMesh(core_axis_name="c", subcore_axis_name="s")
    nc = mesh.num_cores
    groups = _BLK // _LANES

    @functools.partial(
        pl.kernel,
        out_type=jax.ShapeDtypeStruct((num_blocks, 2, _BLK), jnp.float32),
        mesh=mesh,
        scratch_types=[
            pltpu.VMEM((wblocks, 2, _BLK), jnp.float32),
            pltpu.VMEM((_N,), jnp.float32),
        ],
        compiler_params=pltpu.CompilerParams(
            needs_layout_passes=False, use_tc_tiling_on_sc=False
        ),
    )
    def run(x_hbm, f_hbm, out_hbm, buf, ftab):
        wid = lax.axis_index("s") * nc + lax.axis_index("c")
        b0 = wid * wblocks
        pltpu.sync_copy(f_hbm, ftab)
        pltpu.sync_copy(x_hbm.at[pl.ds(b0, wblocks)], buf)

        @plsc.parallel_loop(0, wblocks, 1, unroll=2)
        def step(t):
            for g in range(groups):
                xg = buf[t, 0, pl.ds(g * _LANES, _LANES)]
                s = xg * 128.0 + 128.5
                s = jnp.minimum(jnp.maximum(s, 0.0), 255.0)
                idx = s.astype(jnp.int32)
                fv = plsc.load_gather(ftab, [idx])
                vs = buf.at[t, 1, pl.ds(g * _LANES, _LANES)]
                plsc.addupdate(vs, fv * _DT)
        pltpu.sync_copy(buf, out_hbm.at[pl.ds(b0, wblocks)])

    return run


def kernel(X, force):
    b = X.shape[0]
    xt = jnp.transpose(jnp.reshape(X, (b // _BLK, _BLK, 2)), (0, 2, 1))
    yt = _make_kernel(b // _BLK, 32)(xt, force.astype(jnp.float32))
    return jnp.reshape(jnp.transpose(yt, (0, 2, 1)), (b, 2))
